# baseline (speedup 1.0000x reference)
import functools

import jax
import jax.numpy as jnp
from jax.experimental import pallas as pl
from jax.experimental.pallas import tpu as pltpu


def _body(a_ref, ann_ref, gcb_ref, gk_ref, gr_ref, gb_ref, dw_ref, db_ref,
          o_ref, *, tile_rows):
    i = pl.program_id(1)
    a = a_ref[0]
    ann = ann_ref[0]
    x = jnp.dot(a, ann, preferred_element_type=jnp.float32) + gcb_ref[0]
    h = ann_ref[0, pl.ds(i * tile_rows, tile_rows), :]
    mx = jnp.dot(x, gk_ref[...], preferred_element_type=jnp.float32) + gb_ref[0]
    mi = jnp.dot(h, gr_ref[...], preferred_element_type=jnp.float32) + gb_ref[1]
    c = x.shape[-1]
    z = jax.nn.sigmoid(mx[:, :c] + mi[:, :c])
    r = jax.nn.sigmoid(mx[:, c:2 * c] + mi[:, c:2 * c])
    hh = jnp.tanh(mx[:, 2 * c:] + r * mi[:, 2 * c:])
    h_new = z * h + (1.0 - z) * hh
    o_ref[0] = jnp.dot(h_new, dw_ref[...],
                       preferred_element_type=jnp.float32) + db_ref[...]


def kernel(adjacent, annotations, gc_bias, gru_kernel, gru_recurrent,
           gru_bias, dense_w, dense_b):
    b, n, _ = adjacent.shape
    c = annotations.shape[-1]
    out_ch = dense_w.shape[-1]
    tile_rows = 512

    gc_bias2 = gc_bias.reshape(1, c)
    dense_b2 = dense_b.reshape(1, out_ch)

    grid = (b, n // tile_rows)
    return pl.pallas_call(
        functools.partial(_body, tile_rows=tile_rows),
        grid=grid,
        in_specs=[
            pl.BlockSpec((1, tile_rows, n), lambda bi, i: (bi, i, 0)),
            pl.BlockSpec((1, n, c), lambda bi, i: (bi, 0, 0)),
            pl.BlockSpec((1, c), lambda bi, i: (0, 0)),
            pl.BlockSpec(gru_kernel.shape, lambda bi, i: (0, 0)),
            pl.BlockSpec(gru_recurrent.shape, lambda bi, i: (0, 0)),
            pl.BlockSpec(gru_bias.shape, lambda bi, i: (0, 0)),
            pl.BlockSpec(dense_w.shape, lambda bi, i: (0, 0)),
            pl.BlockSpec((1, out_ch), lambda bi, i: (0, 0)),
        ],
        out_specs=pl.BlockSpec((1, tile_rows, out_ch), lambda bi, i: (bi, i, 0)),
        out_shape=jax.ShapeDtypeStruct((b, n, out_ch), jnp.float32),
        compiler_params=pltpu.CompilerParams(
            dimension_semantics=("arbitrary", "arbitrary"),
        ),
    )(adjacent, annotations, gc_bias2, gru_kernel, gru_recurrent,
      gru_bias, dense_w, dense_b2)


# speedup vs baseline: 1.0559x; 1.0559x over previous
import functools

import jax
import jax.numpy as jnp
from jax.experimental import pallas as pl
from jax.experimental.pallas import tpu as pltpu


def _body(a_ref, ann_ref, gcb_ref, gk_ref, gr_ref, gb_ref, dw_ref, db_ref,
          o_ref, *, tile_rows):
    i = pl.program_id(1)
    a = a_ref[0].astype(jnp.bfloat16)
    ann = ann_ref[0].astype(jnp.bfloat16)
    x = jnp.dot(a, ann, preferred_element_type=jnp.float32) + gcb_ref[0]
    h = ann_ref[0, pl.ds(i * tile_rows, tile_rows), :]
    mx = jnp.dot(x, gk_ref[...], preferred_element_type=jnp.float32) + gb_ref[0]
    mi = jnp.dot(h, gr_ref[...], preferred_element_type=jnp.float32) + gb_ref[1]
    c = x.shape[-1]
    z = jax.nn.sigmoid(mx[:, :c] + mi[:, :c])
    r = jax.nn.sigmoid(mx[:, c:2 * c] + mi[:, c:2 * c])
    hh = jnp.tanh(mx[:, 2 * c:] + r * mi[:, 2 * c:])
    h_new = z * h + (1.0 - z) * hh
    o_ref[0] = jnp.dot(h_new, dw_ref[...],
                       preferred_element_type=jnp.float32) + db_ref[...]


def kernel(adjacent, annotations, gc_bias, gru_kernel, gru_recurrent,
           gru_bias, dense_w, dense_b):
    b, n, _ = adjacent.shape
    c = annotations.shape[-1]
    out_ch = dense_w.shape[-1]
    tile_rows = 1024

    gc_bias2 = gc_bias.reshape(1, c)
    dense_b2 = dense_b.reshape(1, out_ch)

    grid = (b, n // tile_rows)
    return pl.pallas_call(
        functools.partial(_body, tile_rows=tile_rows),
        grid=grid,
        in_specs=[
            pl.BlockSpec((1, tile_rows, n), lambda bi, i: (bi, i, 0)),
            pl.BlockSpec((1, n, c), lambda bi, i: (bi, 0, 0)),
            pl.BlockSpec((1, c), lambda bi, i: (0, 0)),
            pl.BlockSpec(gru_kernel.shape, lambda bi, i: (0, 0)),
            pl.BlockSpec(gru_recurrent.shape, lambda bi, i: (0, 0)),
            pl.BlockSpec(gru_bias.shape, lambda bi, i: (0, 0)),
            pl.BlockSpec(dense_w.shape, lambda bi, i: (0, 0)),
            pl.BlockSpec((1, out_ch), lambda bi, i: (0, 0)),
        ],
        out_specs=pl.BlockSpec((1, tile_rows, out_ch), lambda bi, i: (bi, i, 0)),
        out_shape=jax.ShapeDtypeStruct((b, n, out_ch), jnp.float32),
        compiler_params=pltpu.CompilerParams(
            dimension_semantics=("arbitrary", "arbitrary"),
        ),
    )(adjacent, annotations, gc_bias2, gru_kernel, gru_recurrent,
      gru_bias, dense_w, dense_b2)


# speedup vs baseline: 1.0586x; 1.0025x over previous
import functools

import jax
import jax.numpy as jnp
from jax.experimental import pallas as pl
from jax.experimental.pallas import tpu as pltpu


def _body(a_ref, ann_ref, gcb_ref, gk_ref, gr_ref, gb_ref, dw_ref, db_ref,
          o_ref, *, tile_rows):
    i = pl.program_id(1)
    a = a_ref[0].astype(jnp.bfloat16)
    ann = ann_ref[0].astype(jnp.bfloat16)
    x = jnp.dot(a, ann, preferred_element_type=jnp.float32) + gcb_ref[0]
    h = ann_ref[0, pl.ds(i * tile_rows, tile_rows), :]
    mx = jnp.dot(x, gk_ref[...], preferred_element_type=jnp.float32) + gb_ref[0]
    mi = jnp.dot(h, gr_ref[...], preferred_element_type=jnp.float32) + gb_ref[1]
    c = x.shape[-1]
    z = jax.nn.sigmoid(mx[:, :c] + mi[:, :c])
    r = jax.nn.sigmoid(mx[:, c:2 * c] + mi[:, c:2 * c])
    hh = jnp.tanh(mx[:, 2 * c:] + r * mi[:, 2 * c:])
    h_new = z * h + (1.0 - z) * hh
    o_ref[0] = jnp.dot(h_new, dw_ref[...],
                       preferred_element_type=jnp.float32) + db_ref[...]


def kernel(adjacent, annotations, gc_bias, gru_kernel, gru_recurrent,
           gru_bias, dense_w, dense_b):
    b, n, _ = adjacent.shape
    c = annotations.shape[-1]
    out_ch = dense_w.shape[-1]
    tile_rows = 1024

    gc_bias2 = gc_bias.reshape(1, c)
    dense_b2 = dense_b.reshape(1, out_ch)

    grid = (b, n // tile_rows)
    return pl.pallas_call(
        functools.partial(_body, tile_rows=tile_rows),
        grid=grid,
        in_specs=[
            pl.BlockSpec((1, tile_rows, n), lambda bi, i: (bi, i, 0)),
            pl.BlockSpec((1, n, c), lambda bi, i: (bi, 0, 0)),
            pl.BlockSpec((1, c), lambda bi, i: (0, 0)),
            pl.BlockSpec(gru_kernel.shape, lambda bi, i: (0, 0)),
            pl.BlockSpec(gru_recurrent.shape, lambda bi, i: (0, 0)),
            pl.BlockSpec(gru_bias.shape, lambda bi, i: (0, 0)),
            pl.BlockSpec(dense_w.shape, lambda bi, i: (0, 0)),
            pl.BlockSpec((1, out_ch), lambda bi, i: (0, 0)),
        ],
        out_specs=pl.BlockSpec((1, tile_rows, out_ch), lambda bi, i: (bi, i, 0)),
        out_shape=jax.ShapeDtypeStruct((b, n, out_ch), jnp.float32),
        compiler_params=pltpu.CompilerParams(
            dimension_semantics=("parallel", "parallel"),
        ),
    )(adjacent, annotations, gc_bias2, gru_kernel, gru_recurrent,
      gru_bias, dense_w, dense_b2)
